# baseline (speedup 1.0000x reference)
import functools

import jax
import jax.numpy as jnp
from jax import lax
from jax.experimental import pallas as pl
from jax.experimental.pallas import tpu as pltpu
from jax.experimental.pallas import tpu_sc as plsc

B, V, E = 1024, 100000, 16

NC, NS = 2, 16
NT = NC * NS
CHV = 40
NCHT = V // CHV
NCH_LO = NCHT // NT
NCH_REM = NCHT - NCH_LO * NT
G = 3
NGB = (B // 16) // G

_sc_mesh = plsc.VectorSubcoreMesh(core_axis_name="c", subcore_axis_name="s")


def _bcast(vec, ilane):
    return lax.gather(
        vec, ilane[:, None],
        lax.GatherDimensionNumbers(offset_dims=(), collapsed_slice_dims=(0,),
                                   start_index_map=(0,)),
        slice_sizes=(1,),
        mode=lax.GatherScatterMode.PROMISE_IN_BOUNDS)


@functools.partial(
    pl.kernel,
    out_type=jax.ShapeDtypeStruct((NT, E, B), jnp.float32),
    mesh=_sc_mesh,
    scratch_types=[
        pltpu.VMEM((E, B), jnp.float32),
        pltpu.VMEM((CHV, B), jnp.float32),
        pltpu.VMEM((E, CHV), jnp.float32),
    ],
)
def _sc_embed(emb_c_hbm, x_t_hbm, out_hbm, acc_ref, x_ref, e_ref):
    wid = lax.axis_index("s") * NC + lax.axis_index("c")
    base_ch = wid * NCH_LO + jnp.minimum(wid, NCH_REM)
    n_ch = NCH_LO + jnp.where(wid < NCH_REM, 1, 0)

    zero = jnp.zeros((16,), jnp.float32)

    def _zero(g, carry):
        for e in range(E):
            acc_ref[e, pl.ds(g * 16, 16)] = zero
        return carry

    lax.fori_loop(0, B // 16, _zero, 0)

    def _chunk(c, carry):
        j = base_ch + c
        pltpu.sync_copy(x_t_hbm.at[pl.ds(j * CHV, CHV), :], x_ref)
        pltpu.sync_copy(emb_c_hbm.at[j], e_ref)

        def _gblk(gb, carry2):
            b0 = gb * (G * 16)
            accs = tuple(acc_ref[e, pl.ds(b0 + g * 16, 16)]
                         for e in range(E) for g in range(G))

            def _v(v, accs):
                vb = v // 16
                ilane = jnp.full((16,), v - vb * 16, jnp.int32)
                xvs = [x_ref[v, pl.ds(b0 + g * 16, 16)] for g in range(G)]
                new = []
                i = 0
                for e in range(E):
                    ev = e_ref[e, pl.ds(vb * 16, 16)]
                    s = _bcast(ev, ilane)
                    for g in range(G):
                        new.append(accs[i] + xvs[g] * s)
                        i += 1
                return tuple(new)

            accs = lax.fori_loop(0, CHV, _v, accs)
            i = 0
            for e in range(E):
                for g in range(G):
                    acc_ref[e, pl.ds(b0 + g * 16, 16)] = accs[i]
                    i += 1
            return carry2

        lax.fori_loop(0, NGB, _gblk, 0)

        b0 = NGB * G * 16
        accs = tuple(acc_ref[e, pl.ds(b0, 16)] for e in range(E))

        def _vt(v, accs):
            vb = v // 16
            ilane = jnp.full((16,), v - vb * 16, jnp.int32)
            xv = x_ref[v, pl.ds(b0, 16)]
            new = []
            for e in range(E):
                ev = e_ref[e, pl.ds(vb * 16, 16)]
                s = _bcast(ev, ilane)
                new.append(accs[e] + xv * s)
            return tuple(new)

        accs = lax.fori_loop(0, CHV, _vt, accs)
        for e in range(E):
            acc_ref[e, pl.ds(b0, 16)] = accs[e]
        return carry

    lax.fori_loop(0, n_ch, _chunk, 0)

    pltpu.sync_copy(acc_ref, out_hbm.at[wid])


@jax.jit
def kernel(x, embedding):
    emb_c = embedding.T.reshape(E, NCHT, CHV).transpose(1, 0, 2)
    partials = _sc_embed(emb_c, x.T)
    return partials.sum(axis=0).T
